# baseline (speedup 1.0000x reference)
import functools

import jax
import jax.numpy as jnp
from jax import lax
from jax.experimental import pallas as pl
from jax.experimental.pallas import tpu as pltpu
from jax.experimental.pallas import tpu_sc as plsc

_NUM_FIELDS = 26
_VOCAB = 100000
_EMBED_DIM = 128
_BATCH = 4096
_FPAD = 32
_NC = 2
_NS = 16
_NW = _NC * _NS
_BATCH_PER_W = _BATCH // _NW
_G = 8
_NGROUP = _BATCH_PER_W // _G
_GIDX = _G * _FPAD
_LANES = 16


def _body(xp_hbm, tab_hbm, out_hbm, idx_v, rows_v, gsem):
    wid = lax.axis_index("s") * _NC + lax.axis_index("c")
    wbase = wid * _BATCH_PER_W

    def group_body(grp, carry):
        b0 = wbase + grp * _G
        pltpu.sync_copy(xp_hbm.at[pl.ds(b0 * _FPAD, _GIDX)], idx_v)

        def lane_body(i, carry2):
            q = i * _LANES + lax.iota(jnp.int32, _LANES)
            f = (q % _FPAD) % _NUM_FIELDS
            sl = pl.ds(i * _LANES, _LANES)
            idx_v[sl] = idx_v[sl] + f * _VOCAB
            return carry2

        lax.fori_loop(0, _GIDX // _LANES, lane_body, 0)
        pltpu.async_copy(tab_hbm.at[idx_v], rows_v, gsem).wait()
        for g in range(_G):
            pltpu.sync_copy(rows_v.at[pl.ds(g * _FPAD, _NUM_FIELDS)],
                            out_hbm.at[b0 + g])
        return carry

    lax.fori_loop(0, _NGROUP, group_body, 0)


def kernel(x, tables):
    xp = jnp.pad(x, ((0, 0), (0, _FPAD - _NUM_FIELDS))).reshape(_BATCH * _FPAD)
    tab2d = tables.reshape(_NUM_FIELDS * _VOCAB, _EMBED_DIM)
    mesh = plsc.VectorSubcoreMesh(core_axis_name="c", subcore_axis_name="s")
    k = functools.partial(
        pl.kernel,
        mesh=mesh,
        out_type=jax.ShapeDtypeStruct((_BATCH, _NUM_FIELDS, _EMBED_DIM),
                                      jnp.float32),
        scratch_types=[
            pltpu.VMEM((_GIDX,), jnp.int32),
            pltpu.VMEM((_GIDX, _EMBED_DIM), jnp.float32),
            pltpu.SemaphoreType.DMA,
        ],
    )(_body)
    return k(xp, tab2d)


# speedup vs baseline: 1.0082x; 1.0082x over previous
import functools

import jax
import jax.numpy as jnp
from jax import lax
from jax.experimental import pallas as pl
from jax.experimental.pallas import tpu as pltpu
from jax.experimental.pallas import tpu_sc as plsc

_NUM_FIELDS = 26
_VOCAB = 100000
_EMBED_DIM = 128
_BATCH = 4096
_FPAD = 32
_NC = 2
_NS = 16
_NW = _NC * _NS
_BATCH_PER_W = _BATCH // _NW
_G = 8
_NGROUP = _BATCH_PER_W // _G
_GIDX = _G * _FPAD
_WIDX = _BATCH_PER_W * _FPAD
_LANES = 16


def _body(xp_hbm, tab_hbm, out_hbm, idx_v, rows_a, rows_b, gsa, gsb, wsa, wsb):
    wid = lax.axis_index("s") * _NC + lax.axis_index("c")
    wbase = wid * _BATCH_PER_W

    pltpu.sync_copy(xp_hbm.at[pl.ds(wid * _WIDX, _WIDX)], idx_v)

    def lane_body(i, carry):
        q = i * _LANES + lax.iota(jnp.int32, _LANES)
        f = (q % _FPAD) % _NUM_FIELDS
        sl = pl.ds(i * _LANES, _LANES)
        idx_v[sl] = idx_v[sl] + f * _VOCAB
        return carry

    lax.fori_loop(0, _WIDX // _LANES, lane_body, 0)

    def start_gather(grp, rows, sem):
        return pltpu.async_copy(
            tab_hbm.at[idx_v.at[pl.ds(grp * _GIDX, _GIDX)]], rows, sem)

    def start_wb(grp, rows, sem):
        b0 = wbase + grp * _G
        for g in range(_G):
            pltpu.async_copy(rows.at[pl.ds(g * _FPAD, _NUM_FIELDS)],
                             out_hbm.at[b0 + g], sem)

    def drain_wb(rows, sem):
        for _ in range(_G):
            pltpu.make_async_copy(rows.at[pl.ds(0, _NUM_FIELDS)],
                                  out_hbm.at[0], sem).wait()

    def pair_body(pg, carry):
        g0 = pg * 2
        g1 = g0 + 1

        @pl.when(pg >= 1)
        def _():
            drain_wb(rows_a, wsa)

        ga = start_gather(g0, rows_a, gsa)

        @pl.when(pg >= 1)
        def _():
            drain_wb(rows_b, wsb)

        gb = start_gather(g1, rows_b, gsb)
        ga.wait()
        start_wb(g0, rows_a, wsa)
        gb.wait()
        start_wb(g1, rows_b, wsb)
        return carry

    lax.fori_loop(0, _NGROUP // 2, pair_body, 0)
    drain_wb(rows_a, wsa)
    drain_wb(rows_b, wsb)


def kernel(x, tables):
    xp = jnp.pad(x, ((0, 0), (0, _FPAD - _NUM_FIELDS))).reshape(_BATCH * _FPAD)
    tab2d = tables.reshape(_NUM_FIELDS * _VOCAB, _EMBED_DIM)
    mesh = plsc.VectorSubcoreMesh(core_axis_name="c", subcore_axis_name="s")
    k = functools.partial(
        pl.kernel,
        mesh=mesh,
        out_type=jax.ShapeDtypeStruct((_BATCH, _NUM_FIELDS, _EMBED_DIM),
                                      jnp.float32),
        scratch_types=[
            pltpu.VMEM((_WIDX,), jnp.int32),
            pltpu.VMEM((_GIDX, _EMBED_DIM), jnp.float32),
            pltpu.VMEM((_GIDX, _EMBED_DIM), jnp.float32),
            pltpu.SemaphoreType.DMA,
            pltpu.SemaphoreType.DMA,
            pltpu.SemaphoreType.DMA,
            pltpu.SemaphoreType.DMA,
        ],
    )(_body)
    return k(xp, tab2d)


# speedup vs baseline: 3.0048x; 2.9804x over previous
import functools

import jax
import jax.numpy as jnp
from jax import lax
from jax.experimental import pallas as pl
from jax.experimental.pallas import tpu as pltpu
from jax.experimental.pallas import tpu_sc as plsc

_NUM_FIELDS = 26
_VOCAB = 100000
_EMBED_DIM = 128
_BATCH = 4096
_FPAD = 32
_NC = 2
_NS = 16
_NW = _NC * _NS
_BATCH_PER_W = _BATCH // _NW
_G = 8
_NGROUP = _BATCH_PER_W // _G
_GIDX = _G * _FPAD
_WIDX = _BATCH_PER_W * _FPAD
_LANES = 16


def _body(xp_hbm, tab_hbm, out_hbm, idx_v, rows_a, rows_b, gsa, gsb, wsa, wsb):
    wid = lax.axis_index("s") * _NC + lax.axis_index("c")
    wbase = wid * _BATCH_PER_W

    pltpu.sync_copy(xp_hbm.at[pl.ds(wid * _WIDX, _WIDX)], idx_v)

    def lane_body(i, carry):
        q = i * _LANES + lax.iota(jnp.int32, _LANES)
        f = (q % _FPAD) % _NUM_FIELDS
        sl = pl.ds(i * _LANES, _LANES)
        idx_v[sl] = idx_v[sl] + f * _VOCAB
        return carry

    lax.fori_loop(0, _WIDX // _LANES, lane_body, 0)

    def start_gather(grp, rows, sem):
        for g in range(_G):
            pltpu.async_copy(
                tab_hbm.at[idx_v.at[pl.ds((grp * _G + g) * _FPAD,
                                          _NUM_FIELDS)]],
                rows.at[g], sem)

    def wait_gather(rows, sem):
        for _ in range(_G):
            pltpu.make_async_copy(
                tab_hbm.at[idx_v.at[pl.ds(0, _NUM_FIELDS)]], rows.at[0],
                sem).wait()

    def start_wb(grp, rows, sem):
        b0 = wbase + grp * _G
        pltpu.async_copy(rows, out_hbm.at[pl.ds(b0, _G)], sem)

    def drain_wb(rows, sem):
        pltpu.make_async_copy(rows, out_hbm.at[pl.ds(0, _G)], sem).wait()

    def pair_body(pg, carry):
        g0 = pg * 2
        g1 = g0 + 1

        @pl.when(pg >= 1)
        def _():
            drain_wb(rows_a, wsa)

        start_gather(g0, rows_a, gsa)

        @pl.when(pg >= 1)
        def _():
            drain_wb(rows_b, wsb)

        start_gather(g1, rows_b, gsb)
        wait_gather(rows_a, gsa)
        start_wb(g0, rows_a, wsa)
        wait_gather(rows_b, gsb)
        start_wb(g1, rows_b, wsb)
        return carry

    lax.fori_loop(0, _NGROUP // 2, pair_body, 0)
    drain_wb(rows_a, wsa)
    drain_wb(rows_b, wsb)


def kernel(x, tables):
    xp = jnp.pad(x, ((0, 0), (0, _FPAD - _NUM_FIELDS))).reshape(_BATCH * _FPAD)
    tab2d = tables.reshape(_NUM_FIELDS * _VOCAB, _EMBED_DIM)
    mesh = plsc.VectorSubcoreMesh(core_axis_name="c", subcore_axis_name="s")
    k = functools.partial(
        pl.kernel,
        mesh=mesh,
        out_type=jax.ShapeDtypeStruct((_BATCH, _NUM_FIELDS, _EMBED_DIM),
                                      jnp.float32),
        scratch_types=[
            pltpu.VMEM((_WIDX,), jnp.int32),
            pltpu.VMEM((_G, _NUM_FIELDS, _EMBED_DIM), jnp.float32),
            pltpu.VMEM((_G, _NUM_FIELDS, _EMBED_DIM), jnp.float32),
            pltpu.SemaphoreType.DMA,
            pltpu.SemaphoreType.DMA,
            pltpu.SemaphoreType.DMA,
            pltpu.SemaphoreType.DMA,
        ],
    )(_body)
    return k(xp, tab2d)


# speedup vs baseline: 5.1837x; 1.7252x over previous
import functools

import jax
import jax.numpy as jnp
from jax import lax
from jax.experimental import pallas as pl
from jax.experimental.pallas import tpu as pltpu
from jax.experimental.pallas import tpu_sc as plsc

_NUM_FIELDS = 26
_VOCAB = 100000
_EMBED_DIM = 128
_BATCH = 4096
_BATCH_LOG2 = 12
_TOTAL = _BATCH * _NUM_FIELDS
_NC = 2
_NS = 16
_NW = _NC * _NS
_PER_W = _TOTAL // _NW
_CHUNK = 208
_NCHUNK = _PER_W // _CHUNK
_NBUF = 4
_LANES = 16


def _body(xt_hbm, tab_hbm, out_hbm, idx_v, *scratch):
    rows_refs = scratch[:_NBUF]
    gsems = scratch[_NBUF:2 * _NBUF]
    wsems = scratch[2 * _NBUF:]
    wid = lax.axis_index("s") * _NC + lax.axis_index("c")
    base = wid * _PER_W

    pltpu.sync_copy(xt_hbm.at[pl.ds(base, _PER_W)], idx_v)

    def lane_body(i, carry):
        q = base + i * _LANES + lax.iota(jnp.int32, _LANES)
        f = lax.shift_right_logical(q, _BATCH_LOG2)
        sl = pl.ds(i * _LANES, _LANES)
        idx_v[sl] = idx_v[sl] + f * _VOCAB
        return carry

    lax.fori_loop(0, _PER_W // _LANES, lane_body, 0)

    def start_gather(c, rows, sem):
        return pltpu.async_copy(
            tab_hbm.at[idx_v.at[pl.ds(c * _CHUNK, _CHUNK)]], rows, sem)

    def wb(c, rows, sem):
        return pltpu.async_copy(
            rows, out_hbm.at[pl.ds(base + c * _CHUNK, _CHUNK)], sem)

    def drain_wb(rows, sem):
        pltpu.make_async_copy(rows, out_hbm.at[pl.ds(0, _CHUNK)], sem).wait()

    def round_body(rnd, carry):
        copies = []
        for b in range(_NBUF):
            c = rnd * _NBUF + b

            @pl.when(rnd >= 1)
            def _(b=b):
                drain_wb(rows_refs[b], wsems[b])

            copies.append(start_gather(c, rows_refs[b], gsems[b]))
        for b in range(_NBUF):
            c = rnd * _NBUF + b
            copies[b].wait()
            wb(c, rows_refs[b], wsems[b])
        return carry

    lax.fori_loop(0, _NCHUNK // _NBUF, round_body, 0)
    for b in range(_NBUF):
        drain_wb(rows_refs[b], wsems[b])


def kernel(x, tables):
    xt = x.T.reshape(_TOTAL)
    tab2d = tables.reshape(_NUM_FIELDS * _VOCAB, _EMBED_DIM)
    mesh = plsc.VectorSubcoreMesh(core_axis_name="c", subcore_axis_name="s")
    k = functools.partial(
        pl.kernel,
        mesh=mesh,
        out_type=jax.ShapeDtypeStruct((_TOTAL, _EMBED_DIM), jnp.float32),
        scratch_types=(
            [pltpu.VMEM((_PER_W,), jnp.int32)]
            + [pltpu.VMEM((_CHUNK, _EMBED_DIM), jnp.float32)] * _NBUF
            + [pltpu.SemaphoreType.DMA] * (2 * _NBUF)
        ),
    )(_body)
    out = k(xt, tab2d)
    return out.reshape(_NUM_FIELDS, _BATCH, _EMBED_DIM).transpose(1, 0, 2)


# speedup vs baseline: 5.3594x; 1.0339x over previous
import functools

import jax
import jax.numpy as jnp
from jax import lax
from jax.experimental import pallas as pl
from jax.experimental.pallas import tpu as pltpu
from jax.experimental.pallas import tpu_sc as plsc

_NUM_FIELDS = 26
_VOCAB = 100000
_EMBED_DIM = 128
_BATCH = 4096
_BATCH_LOG2 = 12
_TOTAL = _BATCH * _NUM_FIELDS
_NC = 2
_NS = 16
_NW = _NC * _NS
_PER_W = _TOTAL // _NW
_CHUNK = 416
_NCHUNK = _PER_W // _CHUNK
_NBUF = 2
_LANES = 16


def _body(xt_hbm, tab_hbm, out_hbm, idx_v, *scratch):
    rows_refs = scratch[:_NBUF]
    gsems = scratch[_NBUF:2 * _NBUF]
    wsems = scratch[2 * _NBUF:]
    wid = lax.axis_index("s") * _NC + lax.axis_index("c")
    base = wid * _PER_W

    pltpu.sync_copy(xt_hbm.at[pl.ds(base, _PER_W)], idx_v)

    def lane_body(i, carry):
        q = base + i * _LANES + lax.iota(jnp.int32, _LANES)
        f = lax.shift_right_logical(q, _BATCH_LOG2)
        sl = pl.ds(i * _LANES, _LANES)
        idx_v[sl] = idx_v[sl] + f * _VOCAB
        return carry

    lax.fori_loop(0, _PER_W // _LANES, lane_body, 0)

    def start_gather(c, rows, sem):
        return pltpu.async_copy(
            tab_hbm.at[idx_v.at[pl.ds(c * _CHUNK, _CHUNK)]], rows, sem)

    def wb(c, rows, sem):
        return pltpu.async_copy(
            rows, out_hbm.at[pl.ds(base + c * _CHUNK, _CHUNK)], sem)

    def drain_wb(rows, sem):
        pltpu.make_async_copy(rows, out_hbm.at[pl.ds(0, _CHUNK)], sem).wait()

    def round_body(rnd, carry):
        copies = []
        for b in range(_NBUF):
            c = rnd * _NBUF + b

            @pl.when(rnd >= 1)
            def _(b=b):
                drain_wb(rows_refs[b], wsems[b])

            copies.append(start_gather(c, rows_refs[b], gsems[b]))
        for b in range(_NBUF):
            c = rnd * _NBUF + b
            copies[b].wait()
            wb(c, rows_refs[b], wsems[b])
        return carry

    lax.fori_loop(0, _NCHUNK // _NBUF, round_body, 0)
    for b in range(_NBUF):
        drain_wb(rows_refs[b], wsems[b])


def kernel(x, tables):
    xt = x.T.reshape(_TOTAL)
    tab2d = tables.reshape(_NUM_FIELDS * _VOCAB, _EMBED_DIM)
    mesh = plsc.VectorSubcoreMesh(core_axis_name="c", subcore_axis_name="s")
    k = functools.partial(
        pl.kernel,
        mesh=mesh,
        out_type=jax.ShapeDtypeStruct((_TOTAL, _EMBED_DIM), jnp.float32),
        scratch_types=(
            [pltpu.VMEM((_PER_W,), jnp.int32)]
            + [pltpu.VMEM((_CHUNK, _EMBED_DIM), jnp.float32)] * _NBUF
            + [pltpu.SemaphoreType.DMA] * (2 * _NBUF)
        ),
    )(_body)
    out = k(xt, tab2d)
    return out.reshape(_NUM_FIELDS, _BATCH, _EMBED_DIM).transpose(1, 0, 2)
